# baseline (speedup 1.0000x reference)
import functools

import jax
import jax.numpy as jnp
from jax import lax
from jax.experimental import pallas as pl
from jax.experimental.pallas import tpu as pltpu
from jax.experimental.pallas import tpu_sc as plsc

D_MODEL = 128
D_PAD = 144
NC, NS = 2, 16
NW = NC * NS
GROUP = 128
RING = 5


def _embed_gather(table, x_flat, n_tokens):
    per_w = n_tokens // NW
    n_groups = per_w // GROUP

    mesh = plsc.VectorSubcoreMesh(core_axis_name="c", subcore_axis_name="s")

    @functools.partial(
        pl.kernel,
        mesh=mesh,
        compiler_params=pltpu.CompilerParams(use_tc_tiling_on_sc=False),
        out_type=jax.ShapeDtypeStruct((n_tokens, D_MODEL), jnp.float32),
        scratch_types=[
            pltpu.VMEM((per_w,), jnp.int32),
            pltpu.VMEM((RING, GROUP, D_PAD), jnp.float32),
        ]
        + [pltpu.SemaphoreType.DMA] * (2 * RING),
    )
    def k(x_hbm, tab_hbm, out_hbm, idx_v, rows_v, *sems):
        gsems, ssems = sems[:RING], sems[RING:]
        wid = lax.axis_index("s") * NC + lax.axis_index("c")
        ibase = wid * per_w

        pltpu.sync_copy(x_hbm.at[pl.ds(ibase, per_w)], idx_v)

        def fire_gather(g, r):
            pltpu.async_copy(
                tab_hbm.at[idx_v.at[pl.ds(g * GROUP, GROUP)]],
                rows_v.at[r],
                gsems[r],
            )

        def wait_gather(r):
            pltpu.make_async_copy(
                tab_hbm.at[pl.ds(0, GROUP)], rows_v.at[r], gsems[r]
            ).wait()

        def fire_store(g, r):
            pltpu.async_copy(
                rows_v.at[r, :, pl.ds(0, D_MODEL)],
                out_hbm.at[pl.ds(ibase + g * GROUP, GROUP)],
                ssems[r],
            )

        def wait_store(r):
            pltpu.make_async_copy(
                rows_v.at[r, :, pl.ds(0, D_MODEL)],
                out_hbm.at[pl.ds(0, GROUP)], ssems[r]
            ).wait()

        def step(g, r, refill_g):
            wait_gather(r)
            fire_store(g, r)
            if refill_g is not None:
                rp = (r - 1) % RING
                wait_store(rp)
                fire_gather(refill_g, rp)

        for r in range(RING):
            fire_gather(r, r)

        step(0, 0, None)
        for r in range(1, RING):
            step(r, r, r + RING - 1)

        def body(t, carry):
            for r in range(RING):
                g = t * RING + r
                step(g, r, g + RING - 1)
            return carry

        lax.fori_loop(1, n_groups // RING - 1, body, 0)

        g0 = n_groups - RING
        step(g0, 0, n_groups - 1)
        for r in range(1, RING):
            step(g0 + r, r, None)

        for r in range(RING):
            wait_store(r)

    return k(x_flat, table)


def kernel(x, W_E):
    batch, pos = x.shape
    n_tokens = batch * pos
    x_flat = x.reshape(n_tokens).astype(jnp.int32)
    table = jnp.pad(W_E.T, ((0, 0), (0, D_PAD - D_MODEL)))
    out = _embed_gather(table, x_flat, n_tokens)
    return out.reshape(batch, pos, D_MODEL)


# speedup vs baseline: 1.5097x; 1.5097x over previous
import functools

import jax
import jax.numpy as jnp
from jax import lax
from jax.experimental import pallas as pl
from jax.experimental.pallas import tpu as pltpu
from jax.experimental.pallas import tpu_sc as plsc

D_MODEL = 128
NC, NS = 2, 16
NW = NC * NS
GROUP = 128
RING = 5


def _embed_gather(table, x_flat, n_tokens):
    per_w = n_tokens // NW
    n_groups = per_w // GROUP

    mesh = plsc.VectorSubcoreMesh(core_axis_name="c", subcore_axis_name="s")

    @functools.partial(
        pl.kernel,
        mesh=mesh,
        out_type=jax.ShapeDtypeStruct((n_tokens, D_MODEL), jnp.float32),
        scratch_types=[
            pltpu.VMEM((per_w,), jnp.int32),
            pltpu.VMEM((RING, GROUP, D_MODEL), jnp.float32),
        ]
        + [pltpu.SemaphoreType.DMA] * (2 * RING),
    )
    def k(x_hbm, tab_hbm, out_hbm, idx_v, rows_v, *sems):
        gsems, ssems = sems[:RING], sems[RING:]
        wid = lax.axis_index("s") * NC + lax.axis_index("c")
        ibase = wid * per_w

        pltpu.sync_copy(x_hbm.at[pl.ds(ibase, per_w)], idx_v)

        def fire_gather(g, r):
            pltpu.async_copy(
                tab_hbm.at[idx_v.at[pl.ds(g * GROUP, GROUP)]],
                rows_v.at[r],
                gsems[r],
            )

        def wait_gather(r):
            pltpu.make_async_copy(
                tab_hbm.at[pl.ds(0, GROUP)], rows_v.at[r], gsems[r]
            ).wait()

        def fire_store(g, r):
            pltpu.async_copy(
                rows_v.at[r], out_hbm.at[pl.ds(ibase + g * GROUP, GROUP)],
                ssems[r],
            )

        def wait_store(r):
            pltpu.make_async_copy(
                rows_v.at[r], out_hbm.at[pl.ds(0, GROUP)], ssems[r]
            ).wait()

        def step(g, r, refill_g):
            wait_gather(r)
            fire_store(g, r)
            if refill_g is not None:
                rp = (r - 1) % RING
                wait_store(rp)
                fire_gather(refill_g, rp)

        for r in range(RING):
            fire_gather(r, r)

        step(0, 0, None)
        for r in range(1, RING):
            step(r, r, r + RING - 1)

        def body(t, carry):
            for r in range(RING):
                g = t * RING + r
                step(g, r, g + RING - 1)
            return carry

        lax.fori_loop(1, n_groups // RING - 1, body, 0)

        g0 = n_groups - RING
        step(g0, 0, n_groups - 1)
        for r in range(1, RING):
            step(g0 + r, r, None)

        for r in range(RING):
            wait_store(r)

    return k(x_flat, table)


def kernel(x, W_E):
    batch, pos = x.shape
    n_tokens = batch * pos
    x_flat = x.reshape(n_tokens).astype(jnp.int32)
    table = W_E.T
    out = _embed_gather(table, x_flat, n_tokens)
    return out.reshape(batch, pos, D_MODEL)
